# baseline (speedup 1.0000x reference)
import functools

import jax
import jax.numpy as jnp
from jax import lax
from jax.experimental import pallas as pl
from jax.experimental.pallas import tpu as pltpu
from jax.experimental.pallas import tpu_sc as plsc

NC = 2
NS = 16
L = 16
NW = NC * NS

BN_EPS = 1e-5


def _tc_idx(xt, off_step, B, F, CH):
    def body(xt_ref, out_ref):
        for h in range(2):
            for f in range(F):
                out_ref[pl.ds(h * F * CH + f * CH, CH)] = (
                    xt_ref[f, pl.ds(h * CH, CH)] + (off_step * f))

    return pl.pallas_call(
        body,
        grid=(B // (2 * CH),),
        in_specs=[pl.BlockSpec((F, 2 * CH), lambda c: (0, c))],
        out_specs=pl.BlockSpec((2 * F * CH,), lambda c: (c,)),
        out_shape=jax.ShapeDtypeStruct((B * F,), jnp.int32),
    )(xt)


def _sc_relayout(emb_t, tail2d, V, D):
    BV = 1024
    NB = V // BV
    TAIL = V - NB * BV
    R = V * D // 128

    mesh = plsc.VectorSubcoreMesh(core_axis_name="c", subcore_axis_name="s")

    @functools.partial(
        pl.kernel,
        out_type=jax.ShapeDtypeStruct((R, 128), jnp.float32),
        mesh=mesh,
        scratch_types=[
            pltpu.VMEM((D, BV), jnp.float32),
            pltpu.VMEM((BV * D // 128, 128), jnp.float32),
        ],
        compiler_params=pltpu.CompilerParams(use_tc_tiling_on_sc=True,
                                            needs_layout_passes=False),
    )
    def k(src, tail, dst, in_v, out_v):
        wid = lax.axis_index("s") * NC + lax.axis_index("c")
        lane = lax.iota(jnp.int32, L)
        nblk = (NB - wid + NW - 1) // NW

        def blk(j, _):
            b = wid + j * NW
            pltpu.sync_copy(src.at[:, pl.ds(b * BV, BV)], in_v)

            def row_body(r, _):
                vbase = jnp.broadcast_to(r * 8, (L,))
                for s in range(8):
                    val = plsc.load_gather(in_v, [lane, vbase + s])
                    out_v[r, pl.ds(s * L, L)] = val
                return 0

            lax.fori_loop(0, BV * D // 128, row_body, 0, unroll=8)
            pltpu.sync_copy(out_v, dst.at[pl.ds(b * (BV * D // 128),
                                                BV * D // 128)])
            return 0

        lax.fori_loop(0, nblk, blk, 0, unroll=False)

        if TAIL:
            @pl.when(wid == 0)
            def _tail():
                pltpu.sync_copy(tail, dst.at[pl.ds(NB * (BV * D // 128),
                                                   TAIL * D // 128)])

    return k(emb_t, tail2d)


def _sc_pool(idx_flat, emb_table, lin_flat, B, F, D, CH):
    b_per_w = B // NW
    n_chunks = b_per_w // CH
    GF = CH * F

    mesh = plsc.VectorSubcoreMesh(core_axis_name="c", subcore_axis_name="s")

    @functools.partial(
        pl.kernel,
        out_type=[
            jax.ShapeDtypeStruct((B, D), jnp.float32),
            jax.ShapeDtypeStruct((B,), jnp.float32),
        ],
        mesh=mesh,
        scratch_types=[
            pltpu.VMEM((GF,), jnp.int32),
            pltpu.VMEM((GF, D), jnp.float32),
            pltpu.VMEM((GF,), jnp.float32),
            pltpu.VMEM((CH, D), jnp.float32),
            pltpu.VMEM((CH,), jnp.float32),
            pltpu.SemaphoreType.DMA,
            pltpu.SemaphoreType.DMA,
        ],
        compiler_params=pltpu.CompilerParams(use_tc_tiling_on_sc=False),
    )
    def k(idx_hbm, emb_hbm, lin_hbm, cross_out, lin_out,
          idx_v, rows_v, lin_v, cross_v, linsum_v, sem_e, sem_l):
        wid = lax.axis_index("s") * NC + lax.axis_index("c")
        base = wid * b_per_w

        def chunk_body(c, _):
            row0 = base + c * CH
            pltpu.sync_copy(idx_hbm.at[pl.ds(row0 * F, GF)], idx_v)
            ce = pltpu.async_copy(emb_hbm.at[idx_v], rows_v, sem_e)
            cl = pltpu.async_copy(lin_hbm.at[idx_v], lin_v, sem_l)
            ce.wait()

            def row_body(i, _):
                s = rows_v[i]
                q = s * s
                for f in range(1, F):
                    r = rows_v[f * CH + i]
                    s = s + r
                    q = q + r * r
                cross_v[i] = 0.5 * (s * s - q)
                return 0

            lax.fori_loop(0, CH, row_body, 0, unroll=False)
            cl.wait()

            def grp_body(g, _):
                acc = lin_v[pl.ds(g * L, L)]
                for f in range(1, F):
                    acc = acc + lin_v[pl.ds(f * CH + g * L, L)]
                linsum_v[pl.ds(g * L, L)] = acc
                return 0

            lax.fori_loop(0, CH // L, grp_body, 0, unroll=False)

            pltpu.sync_copy(cross_v, cross_out.at[pl.ds(row0, CH)])
            pltpu.sync_copy(linsum_v, lin_out.at[pl.ds(row0, CH)])
            return 0

        lax.fori_loop(0, n_chunks, chunk_body, 0, unroll=False)

    return k(idx_flat, emb_table, lin_flat)


def _tc_mlp(cross, lin, W1f, b1f, W2t, c0, B, D, H):
    BS = 2048

    def body(cross_ref, lin_ref, w1_ref, b1_ref, w2_ref, c_ref, out_ref):
        h = jnp.dot(cross_ref[...], w1_ref[...],
                    preferred_element_type=jnp.float32) + b1_ref[...]
        h = jnp.maximum(h, 0.0)
        o = jnp.sum(h * w2_ref[...], axis=1)
        out_ref[...] = o + lin_ref[...] + c_ref[0]

    return pl.pallas_call(
        body,
        grid=(B // BS,),
        in_specs=[
            pl.BlockSpec((BS, D), lambda i: (i, 0)),
            pl.BlockSpec((BS,), lambda i: (i,)),
            pl.BlockSpec((D, H), lambda i: (0, 0)),
            pl.BlockSpec((1, H), lambda i: (0, 0)),
            pl.BlockSpec((1, H), lambda i: (0, 0)),
            pl.BlockSpec(memory_space=pltpu.SMEM),
        ],
        out_specs=pl.BlockSpec((BS,), lambda i: (i,)),
        out_shape=jax.ShapeDtypeStruct((B,), jnp.float32),
    )(cross, lin, W1f, b1f, W2t, c0)


def kernel(x, emb_table, lin_table, lin_bias, bn_fm_gamma, bn_fm_beta,
           W1, b1, bn1_gamma, bn1_beta, W2, b2):
    B, F = x.shape
    V, D = emb_table.shape
    H = W1.shape[1]

    idx_flat = _tc_idx(x.T, V // F, B, F, CH=256)

    ntail = V % 1024
    tail2d = emb_table[V - ntail:, :].reshape(ntail * D // 128, 128)
    emb_lin = _sc_relayout(emb_table.T, tail2d, V, D).reshape(V, D)

    cross, lin = _sc_pool(idx_flat, emb_lin, lin_table.reshape(-1), B, F, D,
                          CH=256)

    inv = 1.0 / jnp.sqrt(1.0 + BN_EPS)
    g0 = bn_fm_gamma * inv
    g1 = bn1_gamma * inv
    W1f = (g0[:, None] * W1) * g1[None, :]
    b1f = ((bn_fm_beta @ W1 + b1) * g1 + bn1_beta)[None, :]
    W2t = W2.reshape(1, H)
    c0 = (b2 + lin_bias).reshape(1)

    return _tc_mlp(cross, lin, W1f, b1f, W2t, c0, B, D, H)


# speedup vs baseline: 2.1201x; 2.1201x over previous
import functools

import jax
import jax.numpy as jnp
from jax import lax
from jax.experimental import pallas as pl
from jax.experimental.pallas import tpu as pltpu
from jax.experimental.pallas import tpu_sc as plsc

NC = 2
NS = 16
L = 16
NW = NC * NS

BN_EPS = 1e-5


def _tc_idx(xt, off_step, B, F, CH):
    def body(xt_ref, out_ref):
        for h in range(2):
            for f in range(F):
                out_ref[pl.ds(h * F * CH + f * CH, CH)] = (
                    xt_ref[f, pl.ds(h * CH, CH)] + (off_step * f))

    return pl.pallas_call(
        body,
        grid=(B // (2 * CH),),
        in_specs=[pl.BlockSpec((F, 2 * CH), lambda c: (0, c))],
        out_specs=pl.BlockSpec((2 * F * CH,), lambda c: (c,)),
        out_shape=jax.ShapeDtypeStruct((B * F,), jnp.int32),
    )(xt)


def _sc_relayout(emb_t, tail2d, V, D):
    BV = 1024
    NB = V // BV
    TAIL = V - NB * BV
    R = V * D // 128

    mesh = plsc.VectorSubcoreMesh(core_axis_name="c", subcore_axis_name="s")

    @functools.partial(
        pl.kernel,
        out_type=jax.ShapeDtypeStruct((R, 128), jnp.float32),
        mesh=mesh,
        scratch_types=[
            pltpu.VMEM((D, BV), jnp.float32),
            pltpu.VMEM((BV * D // 128, 128), jnp.float32),
        ],
        compiler_params=pltpu.CompilerParams(use_tc_tiling_on_sc=True,
                                            needs_layout_passes=False),
    )
    def k(src, tail, dst, in_v, out_v):
        wid = lax.axis_index("s") * NC + lax.axis_index("c")
        lane = lax.iota(jnp.int32, L)
        nblk = (NB - wid + NW - 1) // NW
        lanepk = [(lane + k) & 15 for k in range(L)]

        def blk(j, _):
            b = wid + j * NW
            pltpu.sync_copy(src.at[:, pl.ds(b * BV, BV)], in_v)

            def grp_body(g, _):
                vbase = jnp.broadcast_to(g * L, (L,))
                for k in range(L):
                    vv = vbase + lanepk[k]
                    val = plsc.load_gather(in_v, [lane, vv])
                    ro = vv >> 3
                    co = ((vv & 7) << 4) + lane
                    plsc.store_scatter(out_v, [ro, co], val)
                return 0

            lax.fori_loop(0, BV // L, grp_body, 0, unroll=2)
            pltpu.sync_copy(out_v, dst.at[pl.ds(b * (BV * D // 128),
                                                BV * D // 128)])
            return 0

        lax.fori_loop(0, nblk, blk, 0, unroll=False)

        if TAIL:
            @pl.when(wid == 0)
            def _tail():
                pltpu.sync_copy(tail, dst.at[pl.ds(NB * (BV * D // 128),
                                                   TAIL * D // 128)])

    return k(emb_t, tail2d)


def _sc_pool(idx_flat, emb_table, lin_flat, B, F, D, CH):
    b_per_w = B // NW
    n_chunks = b_per_w // CH
    GF = CH * F

    mesh = plsc.VectorSubcoreMesh(core_axis_name="c", subcore_axis_name="s")

    @functools.partial(
        pl.kernel,
        out_type=[
            jax.ShapeDtypeStruct((B, D), jnp.float32),
            jax.ShapeDtypeStruct((B,), jnp.float32),
        ],
        mesh=mesh,
        scratch_types=[
            pltpu.VMEM((GF,), jnp.int32),
            pltpu.VMEM((GF, D), jnp.float32),
            pltpu.VMEM((GF,), jnp.float32),
            pltpu.VMEM((CH, D), jnp.float32),
            pltpu.VMEM((CH,), jnp.float32),
            pltpu.SemaphoreType.DMA,
            pltpu.SemaphoreType.DMA,
        ],
        compiler_params=pltpu.CompilerParams(use_tc_tiling_on_sc=False),
    )
    def k(idx_hbm, emb_hbm, lin_hbm, cross_out, lin_out,
          idx_v, rows_v, lin_v, cross_v, linsum_v, sem_e, sem_l):
        wid = lax.axis_index("s") * NC + lax.axis_index("c")
        base = wid * b_per_w

        def chunk_body(c, _):
            row0 = base + c * CH
            pltpu.sync_copy(idx_hbm.at[pl.ds(row0 * F, GF)], idx_v)
            ce = pltpu.async_copy(emb_hbm.at[idx_v], rows_v, sem_e)
            cl = pltpu.async_copy(lin_hbm.at[idx_v], lin_v, sem_l)
            ce.wait()

            def row_body(i, _):
                s = rows_v[i]
                q = s * s
                for f in range(1, F):
                    r = rows_v[f * CH + i]
                    s = s + r
                    q = q + r * r
                cross_v[i] = 0.5 * (s * s - q)
                return 0

            lax.fori_loop(0, CH, row_body, 0, unroll=False)
            cl.wait()

            def grp_body(g, _):
                acc = lin_v[pl.ds(g * L, L)]
                for f in range(1, F):
                    acc = acc + lin_v[pl.ds(f * CH + g * L, L)]
                linsum_v[pl.ds(g * L, L)] = acc
                return 0

            lax.fori_loop(0, CH // L, grp_body, 0, unroll=False)

            pltpu.sync_copy(cross_v, cross_out.at[pl.ds(row0, CH)])
            pltpu.sync_copy(linsum_v, lin_out.at[pl.ds(row0, CH)])
            return 0

        lax.fori_loop(0, n_chunks, chunk_body, 0, unroll=False)

    return k(idx_flat, emb_table, lin_flat)


def _tc_mlp(cross, lin, W1f, b1f, W2t, c0, B, D, H):
    BS = 2048

    def body(cross_ref, lin_ref, w1_ref, b1_ref, w2_ref, c_ref, out_ref):
        h = jnp.dot(cross_ref[...], w1_ref[...],
                    preferred_element_type=jnp.float32) + b1_ref[...]
        h = jnp.maximum(h, 0.0)
        o = jnp.sum(h * w2_ref[...], axis=1)
        out_ref[...] = o + lin_ref[...] + c_ref[0]

    return pl.pallas_call(
        body,
        grid=(B // BS,),
        in_specs=[
            pl.BlockSpec((BS, D), lambda i: (i, 0)),
            pl.BlockSpec((BS,), lambda i: (i,)),
            pl.BlockSpec((D, H), lambda i: (0, 0)),
            pl.BlockSpec((1, H), lambda i: (0, 0)),
            pl.BlockSpec((1, H), lambda i: (0, 0)),
            pl.BlockSpec(memory_space=pltpu.SMEM),
        ],
        out_specs=pl.BlockSpec((BS,), lambda i: (i,)),
        out_shape=jax.ShapeDtypeStruct((B,), jnp.float32),
    )(cross, lin, W1f, b1f, W2t, c0)


def kernel(x, emb_table, lin_table, lin_bias, bn_fm_gamma, bn_fm_beta,
           W1, b1, bn1_gamma, bn1_beta, W2, b2):
    B, F = x.shape
    V, D = emb_table.shape
    H = W1.shape[1]

    idx_flat = _tc_idx(x.T, V // F, B, F, CH=256)

    ntail = V % 1024
    tail2d = emb_table[V - ntail:, :].reshape(ntail * D // 128, 128)
    emb_lin = _sc_relayout(emb_table.T, tail2d, V, D).reshape(V, D)

    cross, lin = _sc_pool(idx_flat, emb_lin, lin_table.reshape(-1), B, F, D,
                          CH=256)

    inv = 1.0 / jnp.sqrt(1.0 + BN_EPS)
    g0 = bn_fm_gamma * inv
    g1 = bn1_gamma * inv
    W1f = (g0[:, None] * W1) * g1[None, :]
    b1f = ((bn_fm_beta @ W1 + b1) * g1 + bn1_beta)[None, :]
    W2t = W2.reshape(1, H)
    c0 = (b2 + lin_bias).reshape(1)

    return _tc_mlp(cross, lin, W1f, b1f, W2t, c0, B, D, H)


# speedup vs baseline: 2.6533x; 1.2515x over previous
import functools

import jax
import jax.numpy as jnp
from jax import lax
from jax.experimental import pallas as pl
from jax.experimental.pallas import tpu as pltpu
from jax.experimental.pallas import tpu_sc as plsc

NC = 2
NS = 16
L = 16
NW = NC * NS

BN_EPS = 1e-5


def _tc_idx(xt, off_step, B, F, CH):
    def body(xt_ref, out_ref):
        for h in range(2):
            for f in range(F):
                out_ref[pl.ds(h * F * CH + f * CH, CH)] = (
                    xt_ref[f, pl.ds(h * CH, CH)] + (off_step * f))

    return pl.pallas_call(
        body,
        grid=(B // (2 * CH),),
        in_specs=[pl.BlockSpec((F, 2 * CH), lambda c: (0, c))],
        out_specs=pl.BlockSpec((2 * F * CH,), lambda c: (c,)),
        out_shape=jax.ShapeDtypeStruct((B * F,), jnp.int32),
    )(xt)


def _sc_relayout(emb_t, tail2d, V, D):
    BV = 1024
    NB = V // BV
    TAIL = V - NB * BV
    R = V * D // 128

    mesh = plsc.VectorSubcoreMesh(core_axis_name="c", subcore_axis_name="s")

    @functools.partial(
        pl.kernel,
        out_type=jax.ShapeDtypeStruct((R, 128), jnp.float32),
        mesh=mesh,
        scratch_types=[
            pltpu.VMEM((D, BV), jnp.float32),
            pltpu.VMEM((D, BV), jnp.float32),
            pltpu.VMEM((BV * D // 128, 128), jnp.float32),
            pltpu.SemaphoreType.DMA,
            pltpu.SemaphoreType.DMA,
        ],
        compiler_params=pltpu.CompilerParams(use_tc_tiling_on_sc=True,
                                            needs_layout_passes=False),
    )
    def k(src, tail, dst, in_v0, in_v1, out_v, sem0, sem1):
        wid = lax.axis_index("s") * NC + lax.axis_index("c")
        lane = lax.iota(jnp.int32, L)
        nblk = (NB - wid + NW - 1) // NW
        lanepk = [(lane + k) & 15 for k in range(L)]
        cok = [((p & 7) << 4) + lane for p in lanepk]
        ins = (in_v0, in_v1)
        sems = (sem0, sem1)

        def start_in(j, buf):
            @pl.when(j < nblk)
            def _():
                b = wid + j * NW
                pltpu.async_copy(src.at[:, pl.ds(b * BV, BV)], ins[buf],
                                 sems[buf])

        def phase(j, buf):
            @pl.when(j < nblk)
            def _():
                b = wid + j * NW
                pltpu.make_async_copy(src.at[:, pl.ds(b * BV, BV)],
                                      ins[buf], sems[buf]).wait()

                def grp_body(g, _):
                    vbase = jnp.broadcast_to(g * L, (L,))
                    for k in range(L):
                        vv = vbase + lanepk[k]
                        val = plsc.load_gather(ins[buf], [lane, vv])
                        plsc.store_scatter(out_v, [vv >> 3, cok[k]], val)
                    return 0

                lax.fori_loop(0, BV // L, grp_body, 0, unroll=2)
                pltpu.sync_copy(out_v, dst.at[pl.ds(b * (BV * D // 128),
                                                    BV * D // 128)])

        start_in(0, 0)

        def blk2(j2, _):
            j = j2 * 2
            start_in(j + 1, 1)
            phase(j, 0)
            start_in(j + 2, 0)
            phase(j + 1, 1)
            return 0

        lax.fori_loop(0, (NB + NW - 1) // NW // 2 + 1, blk2, 0,
                      unroll=False)

        if TAIL:
            @pl.when(wid == 0)
            def _tail():
                pltpu.sync_copy(tail, dst.at[pl.ds(NB * (BV * D // 128),
                                                   TAIL * D // 128)])

    return k(emb_t, tail2d)


def _sc_pool(idx_flat, emb_table, lin_flat, B, F, D, CH):
    b_per_w = B // NW
    n_chunks = b_per_w // CH
    GF = CH * F

    mesh = plsc.VectorSubcoreMesh(core_axis_name="c", subcore_axis_name="s")

    @functools.partial(
        pl.kernel,
        out_type=[
            jax.ShapeDtypeStruct((B, D), jnp.float32),
            jax.ShapeDtypeStruct((B,), jnp.float32),
        ],
        mesh=mesh,
        scratch_types=[
            pltpu.VMEM((GF,), jnp.int32),
            pltpu.VMEM((GF, D), jnp.float32),
            pltpu.VMEM((GF,), jnp.float32),
            pltpu.VMEM((CH, D), jnp.float32),
            pltpu.VMEM((CH,), jnp.float32),
            pltpu.SemaphoreType.DMA,
            pltpu.SemaphoreType.DMA,
        ],
        compiler_params=pltpu.CompilerParams(use_tc_tiling_on_sc=False),
    )
    def k(idx_hbm, emb_hbm, lin_hbm, cross_out, lin_out,
          idx_v, rows_v, lin_v, cross_v, linsum_v, sem_e, sem_l):
        wid = lax.axis_index("s") * NC + lax.axis_index("c")
        base = wid * b_per_w

        def chunk_body(c, _):
            row0 = base + c * CH
            pltpu.sync_copy(idx_hbm.at[pl.ds(row0 * F, GF)], idx_v)
            ce = pltpu.async_copy(emb_hbm.at[idx_v], rows_v, sem_e)
            cl = pltpu.async_copy(lin_hbm.at[idx_v], lin_v, sem_l)
            ce.wait()

            def row_body(i, _):
                s = rows_v[i]
                q = s * s
                for f in range(1, F):
                    r = rows_v[f * CH + i]
                    s = s + r
                    q = q + r * r
                cross_v[i] = 0.5 * (s * s - q)
                return 0

            lax.fori_loop(0, CH, row_body, 0, unroll=False)
            cl.wait()

            def grp_body(g, _):
                acc = lin_v[pl.ds(g * L, L)]
                for f in range(1, F):
                    acc = acc + lin_v[pl.ds(f * CH + g * L, L)]
                linsum_v[pl.ds(g * L, L)] = acc
                return 0

            lax.fori_loop(0, CH // L, grp_body, 0, unroll=False)

            pltpu.sync_copy(cross_v, cross_out.at[pl.ds(row0, CH)])
            pltpu.sync_copy(linsum_v, lin_out.at[pl.ds(row0, CH)])
            return 0

        lax.fori_loop(0, n_chunks, chunk_body, 0, unroll=False)

    return k(idx_flat, emb_table, lin_flat)


def _tc_mlp(cross, lin, W1f, b1f, W2t, c0, B, D, H):
    BS = 2048

    def body(cross_ref, lin_ref, w1_ref, b1_ref, w2_ref, c_ref, out_ref):
        h = jnp.dot(cross_ref[...], w1_ref[...],
                    preferred_element_type=jnp.float32) + b1_ref[...]
        h = jnp.maximum(h, 0.0)
        o = jnp.sum(h * w2_ref[...], axis=1)
        out_ref[...] = o + lin_ref[...] + c_ref[0]

    return pl.pallas_call(
        body,
        grid=(B // BS,),
        in_specs=[
            pl.BlockSpec((BS, D), lambda i: (i, 0)),
            pl.BlockSpec((BS,), lambda i: (i,)),
            pl.BlockSpec((D, H), lambda i: (0, 0)),
            pl.BlockSpec((1, H), lambda i: (0, 0)),
            pl.BlockSpec((1, H), lambda i: (0, 0)),
            pl.BlockSpec(memory_space=pltpu.SMEM),
        ],
        out_specs=pl.BlockSpec((BS,), lambda i: (i,)),
        out_shape=jax.ShapeDtypeStruct((B,), jnp.float32),
    )(cross, lin, W1f, b1f, W2t, c0)


def kernel(x, emb_table, lin_table, lin_bias, bn_fm_gamma, bn_fm_beta,
           W1, b1, bn1_gamma, bn1_beta, W2, b2):
    B, F = x.shape
    V, D = emb_table.shape
    H = W1.shape[1]

    idx_flat = _tc_idx(x.T, V // F, B, F, CH=256)

    ntail = V % 1024
    tail2d = emb_table[V - ntail:, :].reshape(ntail * D // 128, 128)
    emb_lin = _sc_relayout(emb_table.T, tail2d, V, D).reshape(V, D)

    cross, lin = _sc_pool(idx_flat, emb_lin, lin_table.reshape(-1), B, F, D,
                          CH=256)

    inv = 1.0 / jnp.sqrt(1.0 + BN_EPS)
    g0 = bn_fm_gamma * inv
    g1 = bn1_gamma * inv
    W1f = (g0[:, None] * W1) * g1[None, :]
    b1f = ((bn_fm_beta @ W1 + b1) * g1 + bn1_beta)[None, :]
    W2t = W2.reshape(1, H)
    c0 = (b2 + lin_bias).reshape(1)

    return _tc_mlp(cross, lin, W1f, b1f, W2t, c0, B, D, H)


# speedup vs baseline: 2.9773x; 1.1221x over previous
import functools

import jax
import jax.numpy as jnp
from jax import lax
from jax.experimental import pallas as pl
from jax.experimental.pallas import tpu as pltpu
from jax.experimental.pallas import tpu_sc as plsc

NC = 2
NS = 16
L = 16
NW = NC * NS

BN_EPS = 1e-5


def _tc_idx(xt, off_step, B, F, CH):
    def body(xt_ref, out_ref):
        for h in range(2):
            for f in range(F):
                out_ref[pl.ds(h * F * CH + f * CH, CH)] = (
                    xt_ref[f, pl.ds(h * CH, CH)] + (off_step * f))

    return pl.pallas_call(
        body,
        grid=(B // (2 * CH),),
        in_specs=[pl.BlockSpec((F, 2 * CH), lambda c: (0, c))],
        out_specs=pl.BlockSpec((2 * F * CH,), lambda c: (c,)),
        out_shape=jax.ShapeDtypeStruct((B * F,), jnp.int32),
    )(xt)


def _sc_relayout(emb_t, tail2d, V, D):
    BV = 1024
    NB = V // BV
    TAIL = V - NB * BV
    R = V * D // 128

    mesh = plsc.VectorSubcoreMesh(core_axis_name="c", subcore_axis_name="s")

    @functools.partial(
        pl.kernel,
        out_type=jax.ShapeDtypeStruct((R, 128), jnp.float32),
        mesh=mesh,
        scratch_types=[
            pltpu.VMEM((D, BV), jnp.float32),
            pltpu.VMEM((D, BV), jnp.float32),
            pltpu.VMEM((BV * D // 128, 128), jnp.float32),
            pltpu.VMEM((BV * D // 128, 128), jnp.float32),
            pltpu.SemaphoreType.DMA,
            pltpu.SemaphoreType.DMA,
            pltpu.SemaphoreType.DMA,
            pltpu.SemaphoreType.DMA,
        ],
        compiler_params=pltpu.CompilerParams(use_tc_tiling_on_sc=True,
                                            needs_layout_passes=False),
    )
    def k(src, tail, dst, in_v0, in_v1, out_v0, out_v1,
          sem0, sem1, osem0, osem1):
        wid = lax.axis_index("s") * NC + lax.axis_index("c")
        lane = lax.iota(jnp.int32, L)
        nblk = (NB - wid + NW - 1) // NW
        lanepk = [(lane + k) & 15 for k in range(L)]
        cok = [((p & 7) << 4) + lane for p in lanepk]
        ins = (in_v0, in_v1)
        sems = (sem0, sem1)
        outs = (out_v0, out_v1)
        osems = (osem0, osem1)
        RB = BV * D // 128

        def start_in(j, buf):
            @pl.when(j < nblk)
            def _():
                b = wid + j * NW
                pltpu.async_copy(src.at[:, pl.ds(b * BV, BV)], ins[buf],
                                 sems[buf])

        def phase(j, buf):
            @pl.when(j < nblk)
            def _():
                b = wid + j * NW
                pltpu.make_async_copy(src.at[:, pl.ds(b * BV, BV)],
                                      ins[buf], sems[buf]).wait()

                @pl.when(j >= 2)
                def _wait_out():
                    bo = wid + (j - 2) * NW
                    pltpu.make_async_copy(
                        outs[buf], dst.at[pl.ds(bo * RB, RB)],
                        osems[buf]).wait()

                def grp_body(g, _):
                    vbase = jnp.broadcast_to(g * L, (L,))
                    for k in range(L):
                        vv = vbase + lanepk[k]
                        val = plsc.load_gather(ins[buf], [lane, vv])
                        plsc.store_scatter(outs[buf], [vv >> 3, cok[k]],
                                           val)
                    return 0

                lax.fori_loop(0, BV // L, grp_body, 0, unroll=2)
                pltpu.async_copy(outs[buf], dst.at[pl.ds(b * RB, RB)],
                                 osems[buf])

        start_in(0, 0)

        def blk2(j2, _):
            j = j2 * 2
            start_in(j + 1, 1)
            phase(j, 0)
            start_in(j + 2, 0)
            phase(j + 1, 1)
            return 0

        lax.fori_loop(0, (NB + NW - 1) // NW // 2 + 1, blk2, 0,
                      unroll=False)

        for bfs in range(2):
            @pl.when(nblk > bfs)
            def _drain(bfs=bfs):
                jl = ((nblk - 1 - bfs) // 2) * 2 + bfs
                bo = wid + jl * NW
                pltpu.make_async_copy(outs[bfs],
                                      dst.at[pl.ds(bo * RB, RB)],
                                      osems[bfs]).wait()

        if TAIL:
            @pl.when(wid == 0)
            def _tail():
                pltpu.sync_copy(tail, dst.at[pl.ds(NB * (BV * D // 128),
                                                   TAIL * D // 128)])

    return k(emb_t, tail2d)


def _sc_pool(idx_flat, emb_table, lin_flat, B, F, D, CH):
    b_per_w = B // NW
    n_chunks = b_per_w // CH
    GF = CH * F

    mesh = plsc.VectorSubcoreMesh(core_axis_name="c", subcore_axis_name="s")

    @functools.partial(
        pl.kernel,
        out_type=[
            jax.ShapeDtypeStruct((B, D), jnp.float32),
            jax.ShapeDtypeStruct((B,), jnp.float32),
        ],
        mesh=mesh,
        scratch_types=[
            pltpu.VMEM((GF,), jnp.int32),
            pltpu.VMEM((GF, D), jnp.float32),
            pltpu.VMEM((GF,), jnp.float32),
            pltpu.VMEM((CH, D), jnp.float32),
            pltpu.VMEM((CH,), jnp.float32),
            pltpu.SemaphoreType.DMA,
            pltpu.SemaphoreType.DMA,
        ],
        compiler_params=pltpu.CompilerParams(use_tc_tiling_on_sc=False),
    )
    def k(idx_hbm, emb_hbm, lin_hbm, cross_out, lin_out,
          idx_v, rows_v, lin_v, cross_v, linsum_v, sem_e, sem_l):
        wid = lax.axis_index("s") * NC + lax.axis_index("c")
        base = wid * b_per_w

        def chunk_body(c, _):
            row0 = base + c * CH
            pltpu.sync_copy(idx_hbm.at[pl.ds(row0 * F, GF)], idx_v)
            ce = pltpu.async_copy(emb_hbm.at[idx_v], rows_v, sem_e)
            cl = pltpu.async_copy(lin_hbm.at[idx_v], lin_v, sem_l)
            ce.wait()

            def row_body(i, _):
                s = rows_v[i]
                q = s * s
                for f in range(1, F):
                    r = rows_v[f * CH + i]
                    s = s + r
                    q = q + r * r
                cross_v[i] = 0.5 * (s * s - q)
                return 0

            lax.fori_loop(0, CH, row_body, 0, unroll=False)
            cl.wait()

            def grp_body(g, _):
                acc = lin_v[pl.ds(g * L, L)]
                for f in range(1, F):
                    acc = acc + lin_v[pl.ds(f * CH + g * L, L)]
                linsum_v[pl.ds(g * L, L)] = acc
                return 0

            lax.fori_loop(0, CH // L, grp_body, 0, unroll=False)

            pltpu.sync_copy(cross_v, cross_out.at[pl.ds(row0, CH)])
            pltpu.sync_copy(linsum_v, lin_out.at[pl.ds(row0, CH)])
            return 0

        lax.fori_loop(0, n_chunks, chunk_body, 0, unroll=False)

    return k(idx_flat, emb_table, lin_flat)


def _tc_mlp(cross, lin, W1f, b1f, W2t, c0, B, D, H):
    BS = 2048

    def body(cross_ref, lin_ref, w1_ref, b1_ref, w2_ref, c_ref, out_ref):
        h = jnp.dot(cross_ref[...], w1_ref[...],
                    preferred_element_type=jnp.float32) + b1_ref[...]
        h = jnp.maximum(h, 0.0)
        o = jnp.sum(h * w2_ref[...], axis=1)
        out_ref[...] = o + lin_ref[...] + c_ref[0]

    return pl.pallas_call(
        body,
        grid=(B // BS,),
        in_specs=[
            pl.BlockSpec((BS, D), lambda i: (i, 0)),
            pl.BlockSpec((BS,), lambda i: (i,)),
            pl.BlockSpec((D, H), lambda i: (0, 0)),
            pl.BlockSpec((1, H), lambda i: (0, 0)),
            pl.BlockSpec((1, H), lambda i: (0, 0)),
            pl.BlockSpec(memory_space=pltpu.SMEM),
        ],
        out_specs=pl.BlockSpec((BS,), lambda i: (i,)),
        out_shape=jax.ShapeDtypeStruct((B,), jnp.float32),
    )(cross, lin, W1f, b1f, W2t, c0)


def kernel(x, emb_table, lin_table, lin_bias, bn_fm_gamma, bn_fm_beta,
           W1, b1, bn1_gamma, bn1_beta, W2, b2):
    B, F = x.shape
    V, D = emb_table.shape
    H = W1.shape[1]

    idx_flat = _tc_idx(x.T, V // F, B, F, CH=256)

    ntail = V % 1024
    tail2d = emb_table[V - ntail:, :].reshape(ntail * D // 128, 128)
    emb_lin = _sc_relayout(emb_table.T, tail2d, V, D).reshape(V, D)

    cross, lin = _sc_pool(idx_flat, emb_lin, lin_table.reshape(-1), B, F, D,
                          CH=256)

    inv = 1.0 / jnp.sqrt(1.0 + BN_EPS)
    g0 = bn_fm_gamma * inv
    g1 = bn1_gamma * inv
    W1f = (g0[:, None] * W1) * g1[None, :]
    b1f = ((bn_fm_beta @ W1 + b1) * g1 + bn1_beta)[None, :]
    W2t = W2.reshape(1, H)
    c0 = (b2 + lin_bias).reshape(1)

    return _tc_mlp(cross, lin, W1f, b1f, W2t, c0, B, D, H)


# speedup vs baseline: 2.9811x; 1.0013x over previous
import functools

import jax
import jax.numpy as jnp
from jax import lax
from jax.experimental import pallas as pl
from jax.experimental.pallas import tpu as pltpu
from jax.experimental.pallas import tpu_sc as plsc

NC = 2
NS = 16
L = 16
NW = NC * NS

BN_EPS = 1e-5


def _tc_idx(xt, off_step, B, F, CH):
    def body(xt_ref, out_ref):
        for h in range(2):
            for f in range(F):
                out_ref[pl.ds(h * F * CH + f * CH, CH)] = (
                    xt_ref[f, pl.ds(h * CH, CH)] + (off_step * f))

    return pl.pallas_call(
        body,
        grid=(B // (2 * CH),),
        in_specs=[pl.BlockSpec((F, 2 * CH), lambda c: (0, c))],
        out_specs=pl.BlockSpec((2 * F * CH,), lambda c: (c,)),
        out_shape=jax.ShapeDtypeStruct((B * F,), jnp.int32),
    )(xt)


def _sc_relayout(emb_t, tail2d, V, D):
    BV = 1024
    NB = V // BV
    TAIL = V - NB * BV
    R = V * D // 128

    mesh = plsc.VectorSubcoreMesh(core_axis_name="c", subcore_axis_name="s")

    @functools.partial(
        pl.kernel,
        out_type=jax.ShapeDtypeStruct((R, 128), jnp.float32),
        mesh=mesh,
        scratch_types=[
            pltpu.VMEM((D, BV), jnp.float32),
            pltpu.VMEM((D, BV), jnp.float32),
            pltpu.VMEM((BV * D // 128, 128), jnp.float32),
            pltpu.VMEM((BV * D // 128, 128), jnp.float32),
            pltpu.SemaphoreType.DMA,
            pltpu.SemaphoreType.DMA,
            pltpu.SemaphoreType.DMA,
            pltpu.SemaphoreType.DMA,
        ],
        compiler_params=pltpu.CompilerParams(use_tc_tiling_on_sc=True,
                                            needs_layout_passes=False),
    )
    def k(src, tail, dst, in_v0, in_v1, out_v0, out_v1,
          sem0, sem1, osem0, osem1):
        wid = lax.axis_index("s") * NC + lax.axis_index("c")
        lane = lax.iota(jnp.int32, L)
        nblk = (NB - wid + NW - 1) // NW
        lanepk = [(lane + k) & 15 for k in range(L)]
        cok = [((p & 7) << 4) + lane for p in lanepk]
        ins = (in_v0, in_v1)
        sems = (sem0, sem1)
        outs = (out_v0, out_v1)
        osems = (osem0, osem1)
        RB = BV * D // 128

        def start_in(j, buf):
            @pl.when(j < nblk)
            def _():
                b = wid + j * NW
                pltpu.async_copy(src.at[:, pl.ds(b * BV, BV)], ins[buf],
                                 sems[buf])

        def phase(j, buf):
            @pl.when(j < nblk)
            def _():
                b = wid + j * NW
                pltpu.make_async_copy(src.at[:, pl.ds(b * BV, BV)],
                                      ins[buf], sems[buf]).wait()

                @pl.when(j >= 2)
                def _wait_out():
                    bo = wid + (j - 2) * NW
                    pltpu.make_async_copy(
                        outs[buf], dst.at[pl.ds(bo * RB, RB)],
                        osems[buf]).wait()

                def grp_body(g, _):
                    vbase = jnp.broadcast_to(g * L, (L,))
                    for k in range(L):
                        vv = vbase + lanepk[k]
                        val = plsc.load_gather(ins[buf], [lane, vv])
                        plsc.store_scatter(outs[buf], [vv >> 3, cok[k]],
                                           val)
                    return 0

                lax.fori_loop(0, BV // L, grp_body, 0, unroll=4)
                pltpu.async_copy(outs[buf], dst.at[pl.ds(b * RB, RB)],
                                 osems[buf])

        start_in(0, 0)

        def blk2(j2, _):
            j = j2 * 2
            start_in(j + 1, 1)
            phase(j, 0)
            start_in(j + 2, 0)
            phase(j + 1, 1)
            return 0

        lax.fori_loop(0, (NB + NW - 1) // NW // 2 + 1, blk2, 0,
                      unroll=False)

        for bfs in range(2):
            @pl.when(nblk > bfs)
            def _drain(bfs=bfs):
                jl = ((nblk - 1 - bfs) // 2) * 2 + bfs
                bo = wid + jl * NW
                pltpu.make_async_copy(outs[bfs],
                                      dst.at[pl.ds(bo * RB, RB)],
                                      osems[bfs]).wait()

        if TAIL:
            @pl.when(wid == 0)
            def _tail():
                pltpu.sync_copy(tail, dst.at[pl.ds(NB * (BV * D // 128),
                                                   TAIL * D // 128)])

    return k(emb_t, tail2d)


def _sc_pool(idx_flat, emb_table, lin_flat, B, F, D, CH):
    b_per_w = B // NW
    n_chunks = b_per_w // CH
    GF = CH * F

    mesh = plsc.VectorSubcoreMesh(core_axis_name="c", subcore_axis_name="s")

    @functools.partial(
        pl.kernel,
        out_type=[
            jax.ShapeDtypeStruct((B, D), jnp.float32),
            jax.ShapeDtypeStruct((B,), jnp.float32),
        ],
        mesh=mesh,
        scratch_types=[
            pltpu.VMEM((GF,), jnp.int32),
            pltpu.VMEM((GF, D), jnp.float32),
            pltpu.VMEM((GF,), jnp.float32),
            pltpu.VMEM((CH, D), jnp.float32),
            pltpu.VMEM((CH,), jnp.float32),
            pltpu.SemaphoreType.DMA,
            pltpu.SemaphoreType.DMA,
        ],
        compiler_params=pltpu.CompilerParams(use_tc_tiling_on_sc=False),
    )
    def k(idx_hbm, emb_hbm, lin_hbm, cross_out, lin_out,
          idx_v, rows_v, lin_v, cross_v, linsum_v, sem_e, sem_l):
        wid = lax.axis_index("s") * NC + lax.axis_index("c")
        base = wid * b_per_w

        def chunk_body(c, _):
            row0 = base + c * CH
            pltpu.sync_copy(idx_hbm.at[pl.ds(row0 * F, GF)], idx_v)
            ce = pltpu.async_copy(emb_hbm.at[idx_v], rows_v, sem_e)
            cl = pltpu.async_copy(lin_hbm.at[idx_v], lin_v, sem_l)
            ce.wait()

            def row_body(i, _):
                s = rows_v[i]
                q = s * s
                for f in range(1, F):
                    r = rows_v[f * CH + i]
                    s = s + r
                    q = q + r * r
                cross_v[i] = 0.5 * (s * s - q)
                return 0

            lax.fori_loop(0, CH, row_body, 0, unroll=False)
            cl.wait()

            def grp_body(g, _):
                acc = lin_v[pl.ds(g * L, L)]
                for f in range(1, F):
                    acc = acc + lin_v[pl.ds(f * CH + g * L, L)]
                linsum_v[pl.ds(g * L, L)] = acc
                return 0

            lax.fori_loop(0, CH // L, grp_body, 0, unroll=False)

            pltpu.sync_copy(cross_v, cross_out.at[pl.ds(row0, CH)])
            pltpu.sync_copy(linsum_v, lin_out.at[pl.ds(row0, CH)])
            return 0

        lax.fori_loop(0, n_chunks, chunk_body, 0, unroll=False)

    return k(idx_flat, emb_table, lin_flat)


def _tc_mlp(cross, lin, W1f, b1f, W2t, c0, B, D, H):
    BS = 2048

    def body(cross_ref, lin_ref, w1_ref, b1_ref, w2_ref, c_ref, out_ref):
        h = jnp.dot(cross_ref[...], w1_ref[...],
                    preferred_element_type=jnp.float32) + b1_ref[...]
        h = jnp.maximum(h, 0.0)
        o = jnp.sum(h * w2_ref[...], axis=1)
        out_ref[...] = o + lin_ref[...] + c_ref[0]

    return pl.pallas_call(
        body,
        grid=(B // BS,),
        in_specs=[
            pl.BlockSpec((BS, D), lambda i: (i, 0)),
            pl.BlockSpec((BS,), lambda i: (i,)),
            pl.BlockSpec((D, H), lambda i: (0, 0)),
            pl.BlockSpec((1, H), lambda i: (0, 0)),
            pl.BlockSpec((1, H), lambda i: (0, 0)),
            pl.BlockSpec(memory_space=pltpu.SMEM),
        ],
        out_specs=pl.BlockSpec((BS,), lambda i: (i,)),
        out_shape=jax.ShapeDtypeStruct((B,), jnp.float32),
    )(cross, lin, W1f, b1f, W2t, c0)


def kernel(x, emb_table, lin_table, lin_bias, bn_fm_gamma, bn_fm_beta,
           W1, b1, bn1_gamma, bn1_beta, W2, b2):
    B, F = x.shape
    V, D = emb_table.shape
    H = W1.shape[1]

    idx_flat = _tc_idx(x.T, V // F, B, F, CH=256)

    ntail = V % 1024
    tail2d = emb_table[V - ntail:, :].reshape(ntail * D // 128, 128)
    emb_lin = _sc_relayout(emb_table.T, tail2d, V, D).reshape(V, D)

    cross, lin = _sc_pool(idx_flat, emb_lin, lin_table.reshape(-1), B, F, D,
                          CH=256)

    inv = 1.0 / jnp.sqrt(1.0 + BN_EPS)
    g0 = bn_fm_gamma * inv
    g1 = bn1_gamma * inv
    W1f = (g0[:, None] * W1) * g1[None, :]
    b1f = ((bn_fm_beta @ W1 + b1) * g1 + bn1_beta)[None, :]
    W2t = W2.reshape(1, H)
    c0 = (b2 + lin_bias).reshape(1)

    return _tc_mlp(cross, lin, W1f, b1f, W2t, c0, B, D, H)
